# baseline (speedup 1.0000x reference)
import jax
import jax.numpy as jnp
from jax.experimental import pallas as pl
from jax.experimental.pallas import tpu as pltpu

D = 1024
DFF = 2816
E = 8
T = 32
BF = 256
NBF = DFF // BF


def _moe_body(x_ref, gw_ref, wg_ref, wu_ref, wd_ref, out_ref, comb_ref, act_ref):
    e = pl.program_id(0)
    j = pl.program_id(1)

    @pl.when((e == 0) & (j == 0))
    def _router():
        xv = x_ref[...]
        logits = jax.lax.dot_general(
            xv, gw_ref[...], (((1,), (1,)), ((), ())),
            preferred_element_type=jnp.float32)
        p = jnp.exp(logits - jnp.max(logits, axis=1, keepdims=True))
        idx = jax.lax.broadcasted_iota(jnp.int32, (T, E), 1)
        m1 = jnp.max(p, axis=1, keepdims=True)
        i1 = jnp.min(jnp.where(p == m1, idx, E), axis=1, keepdims=True)
        mask1 = idx == i1
        p_wo = jnp.where(mask1, -jnp.inf, p)
        m2 = jnp.max(p_wo, axis=1, keepdims=True)
        i2 = jnp.min(jnp.where(p_wo == m2, idx, E), axis=1, keepdims=True)
        mask = mask1 | (idx == i2)
        pm = jnp.where(mask, p, 0.0)
        comb_ref[...] = pm / jnp.sum(pm, axis=1, keepdims=True)
        out_ref[...] = jnp.zeros_like(out_ref)

    xv = x_ref[...].astype(jnp.bfloat16)
    g = jax.lax.dot_general(
        xv, wg_ref[0].astype(jnp.bfloat16), (((1,), (1,)), ((), ())),
        preferred_element_type=jnp.float32)
    u = jax.lax.dot_general(
        xv, wu_ref[0].astype(jnp.bfloat16), (((1,), (1,)), ((), ())),
        preferred_element_type=jnp.float32)
    act_ref[j] = (g * jax.nn.sigmoid(g) * u).astype(jnp.bfloat16)

    @pl.when(j == NBF - 1)
    def _down():
        wd = wd_ref[0]
        part = jnp.zeros((T, D), dtype=jnp.float32)
        for k in range(NBF):
            part += jax.lax.dot_general(
                act_ref[k], wd[:, k * BF:(k + 1) * BF].astype(jnp.bfloat16),
                (((1,), (1,)), ((), ())),
                preferred_element_type=jnp.float32)
        sel = (jax.lax.broadcasted_iota(jnp.int32, (E, 1), 0) == e).astype(
            jnp.float32)
        scale = jax.lax.dot_general(
            comb_ref[...], sel, (((1,), (0,)), ((), ())),
            preferred_element_type=jnp.float32)
        out_ref[...] += part * scale


def kernel(x, gate_w, Wg, Wu, Wd):
    x2d = x.reshape(T, D)
    out = pl.pallas_call(
        _moe_body,
        grid=(E, NBF),
        in_specs=[
            pl.BlockSpec((T, D), lambda e, j: (0, 0)),
            pl.BlockSpec((E, D), lambda e, j: (0, 0)),
            pl.BlockSpec((1, BF, D), lambda e, j: (e, j, 0)),
            pl.BlockSpec((1, BF, D), lambda e, j: (e, j, 0)),
            pl.BlockSpec((1, D, DFF), lambda e, j: (e, 0, 0)),
        ],
        out_specs=pl.BlockSpec((T, D), lambda e, j: (0, 0)),
        out_shape=jax.ShapeDtypeStruct((T, D), jnp.float32),
        scratch_shapes=[
            pltpu.VMEM((T, E), jnp.float32),
            pltpu.VMEM((NBF, T, BF), jnp.bfloat16),
        ],
    )(x2d, gate_w, Wg, Wu, Wd)
    return out.reshape(x.shape)


# speedup vs baseline: 1.3793x; 1.3793x over previous
import jax
import jax.numpy as jnp
from jax.experimental import pallas as pl
from jax.experimental.pallas import tpu as pltpu

D = 1024
DFF = 2816
E = 8
T = 32
BF = 1408
NBF = DFF // BF


def _moe_body(x_ref, gw_ref, wg_ref, wu_ref, wd_ref, out_ref, comb_ref, act_ref):
    e = pl.program_id(0)
    j = pl.program_id(1)

    @pl.when((e == 0) & (j == 0))
    def _router():
        xv = x_ref[...]
        logits = jax.lax.dot_general(
            xv, gw_ref[...], (((1,), (1,)), ((), ())),
            preferred_element_type=jnp.float32)
        p = jnp.exp(logits - jnp.max(logits, axis=1, keepdims=True))
        idx = jax.lax.broadcasted_iota(jnp.int32, (T, E), 1)
        m1 = jnp.max(p, axis=1, keepdims=True)
        i1 = jnp.min(jnp.where(p == m1, idx, E), axis=1, keepdims=True)
        mask1 = idx == i1
        p_wo = jnp.where(mask1, -jnp.inf, p)
        m2 = jnp.max(p_wo, axis=1, keepdims=True)
        i2 = jnp.min(jnp.where(p_wo == m2, idx, E), axis=1, keepdims=True)
        mask = mask1 | (idx == i2)
        pm = jnp.where(mask, p, 0.0)
        comb_ref[...] = pm / jnp.sum(pm, axis=1, keepdims=True)
        out_ref[...] = jnp.zeros_like(out_ref)

    xv = x_ref[...].astype(jnp.bfloat16)
    g = jax.lax.dot_general(
        xv, wg_ref[0].astype(jnp.bfloat16), (((1,), (1,)), ((), ())),
        preferred_element_type=jnp.float32)
    u = jax.lax.dot_general(
        xv, wu_ref[0].astype(jnp.bfloat16), (((1,), (1,)), ((), ())),
        preferred_element_type=jnp.float32)
    act_ref[j] = (g * jax.nn.sigmoid(g) * u).astype(jnp.bfloat16)

    @pl.when(j == NBF - 1)
    def _down():
        wd = wd_ref[0]
        part = jnp.zeros((T, D), dtype=jnp.float32)
        for k in range(NBF):
            part += jax.lax.dot_general(
                act_ref[k], wd[:, k * BF:(k + 1) * BF].astype(jnp.bfloat16),
                (((1,), (1,)), ((), ())),
                preferred_element_type=jnp.float32)
        sel = (jax.lax.broadcasted_iota(jnp.int32, (E, 1), 0) == e).astype(
            jnp.float32)
        scale = jax.lax.dot_general(
            comb_ref[...], sel, (((1,), (0,)), ((), ())),
            preferred_element_type=jnp.float32)
        out_ref[...] += part * scale


def kernel(x, gate_w, Wg, Wu, Wd):
    x2d = x.reshape(T, D)
    out = pl.pallas_call(
        _moe_body,
        grid=(E, NBF),
        in_specs=[
            pl.BlockSpec((T, D), lambda e, j: (0, 0)),
            pl.BlockSpec((E, D), lambda e, j: (0, 0)),
            pl.BlockSpec((1, BF, D), lambda e, j: (e, j, 0)),
            pl.BlockSpec((1, BF, D), lambda e, j: (e, j, 0)),
            pl.BlockSpec((1, D, DFF), lambda e, j: (e, 0, 0)),
        ],
        out_specs=pl.BlockSpec((T, D), lambda e, j: (0, 0)),
        out_shape=jax.ShapeDtypeStruct((T, D), jnp.float32),
        scratch_shapes=[
            pltpu.VMEM((T, E), jnp.float32),
            pltpu.VMEM((NBF, T, BF), jnp.bfloat16),
        ],
    )(x2d, gate_w, Wg, Wu, Wd)
    return out.reshape(x.shape)


# speedup vs baseline: 1.4309x; 1.0374x over previous
import jax
import jax.numpy as jnp
from jax.experimental import pallas as pl
from jax.experimental.pallas import tpu as pltpu

D = 1024
DFF = 2816
E = 8
T = 32
BF = 1408
NBF = DFF // BF


def _moe_body(x_ref, gw_ref, wg_ref, wu_ref, wd_ref, out_ref, comb_ref):
    e = pl.program_id(0)
    j = pl.program_id(1)

    @pl.when((e == 0) & (j == 0))
    def _router():
        xv = x_ref[...]
        logits = jax.lax.dot_general(
            xv, gw_ref[...], (((1,), (1,)), ((), ())),
            preferred_element_type=jnp.float32)
        p = jnp.exp(logits - jnp.max(logits, axis=1, keepdims=True))
        idx = jax.lax.broadcasted_iota(jnp.int32, (T, E), 1)
        m1 = jnp.max(p, axis=1, keepdims=True)
        i1 = jnp.min(jnp.where(p == m1, idx, E), axis=1, keepdims=True)
        mask1 = idx == i1
        p_wo = jnp.where(mask1, -jnp.inf, p)
        m2 = jnp.max(p_wo, axis=1, keepdims=True)
        i2 = jnp.min(jnp.where(p_wo == m2, idx, E), axis=1, keepdims=True)
        mask = mask1 | (idx == i2)
        pm = jnp.where(mask, p, 0.0)
        comb_ref[...] = pm / jnp.sum(pm, axis=1, keepdims=True)
        out_ref[...] = jnp.zeros_like(out_ref)

    xv = x_ref[...].astype(jnp.bfloat16)
    g = jax.lax.dot_general(
        xv, wg_ref[0].astype(jnp.bfloat16), (((1,), (1,)), ((), ())),
        preferred_element_type=jnp.float32)
    u = jax.lax.dot_general(
        xv, wu_ref[0].astype(jnp.bfloat16), (((1,), (1,)), ((), ())),
        preferred_element_type=jnp.float32)
    act = (g * jax.nn.sigmoid(g) * u).astype(jnp.bfloat16)
    part = jax.lax.dot_general(
        act, wd_ref[0].astype(jnp.bfloat16), (((1,), (1,)), ((), ())),
        preferred_element_type=jnp.float32)
    sel = (jax.lax.broadcasted_iota(jnp.int32, (E, 1), 0) == e).astype(
        jnp.float32)
    scale = jax.lax.dot_general(
        comb_ref[...], sel, (((1,), (0,)), ((), ())),
        preferred_element_type=jnp.float32)
    out_ref[...] += part * scale


def kernel(x, gate_w, Wg, Wu, Wd):
    x2d = x.reshape(T, D)
    out = pl.pallas_call(
        _moe_body,
        grid=(E, NBF),
        in_specs=[
            pl.BlockSpec((T, D), lambda e, j: (0, 0)),
            pl.BlockSpec((E, D), lambda e, j: (0, 0)),
            pl.BlockSpec((1, BF, D), lambda e, j: (e, j, 0)),
            pl.BlockSpec((1, BF, D), lambda e, j: (e, j, 0)),
            pl.BlockSpec((1, D, BF), lambda e, j: (e, 0, j)),
        ],
        out_specs=pl.BlockSpec((T, D), lambda e, j: (0, 0)),
        out_shape=jax.ShapeDtypeStruct((T, D), jnp.float32),
        scratch_shapes=[pltpu.VMEM((T, E), jnp.float32)],
    )(x2d, gate_w, Wg, Wu, Wd)
    return out.reshape(x.shape)
